# baseline (speedup 1.0000x reference)
import numpy as np
import jax
import jax.numpy as jnp
from jax.experimental import pallas as pl
from jax.experimental.pallas import tpu as pltpu

ND = 256
ND2 = 512
NV = 30
NE = 16
BSZ = 256
CORE = 45
CPAD = 48
MHCL = 34
W0 = 42
WL = 37
LP = 79
NSHARE = 42
BT = 128
NB = BSZ // BT


def _pe_table(d, length):
    pos = np.arange(length)[:, None].astype(np.float32)
    i = np.arange(d)[None, :].astype(np.float32)
    ang = pos / np.power(10000.0, (2.0 * np.floor(i / 2.0)) / d)
    pe = np.zeros((length, d), np.float32)
    pe[:, 0::2] = np.sin(ang[:, 0::2])
    pe[:, 1::2] = np.cos(ang[:, 1::2])
    return pe


_PE79 = _pe_table(ND, LP)
_PE_CORE = np.zeros((CPAD, ND), np.float32)
_PE_CORE[:CORE] = _PE79[:CORE]
_PE_WIN = np.ascontiguousarray(_PE79[W0:LP])


def _dot(a, b):
    return jax.lax.dot_general(a, b, (((1,), (0,)), ((), ())),
                               preferred_element_type=jnp.float32)


def _embed(tok2, tok3, embpe):
    n, bt = tok3.shape
    rows = n * bt
    k = NV + n
    lane = jax.lax.broadcasted_iota(jnp.int32, (rows, k), 1)
    pos = jax.lax.broadcasted_iota(jnp.int32, (rows, 1), 0) // bt + NV
    oh = ((lane == tok2) | (lane == pos)).astype(jnp.float32)
    feat2 = _dot(oh, embpe)
    valid3 = (tok3 != 0).astype(jnp.float32)[:, :, None]
    feat = feat2.reshape(n, bt, ND) * valid3
    return feat, valid3


def _body(tokA2_ref, tokA3_ref, tokB2_ref, tokB3_ref, tok0_ref, bo_ref,
          embpeA_ref, wcat_ref, bc_ref, wo_ref, embpeB_ref,
          out_ref, wsc_ref, halo_ref):
    b = pl.program_id(0)
    e = pl.program_id(1)
    wcat = wcat_ref[...]
    bc = bc_ref[...]

    @pl.when(e == 0)
    def _core():
        tok2 = tokA2_ref[0]
        tok3 = tokA3_ref[0]
        feat, valid3 = _embed(tok2, tok3, embpeA_ref[...])
        rows = CPAD * BT
        halo_ref[0] = feat[41]
        z1 = jnp.zeros((1, BT, ND), jnp.float32)
        h = feat
        for layer in range(3):
            left = jnp.concatenate([z1, h[:-1]], axis=0)
            right = jnp.concatenate([h[1:], z1], axis=0)
            mi = left + right
            cat = jnp.concatenate([h.reshape(rows, ND), mi.reshape(rows, ND)],
                                  axis=1)
            pre = _dot(cat, wcat) + bc
            h = jnp.maximum(pre, 0.0).reshape(CPAD, BT, ND) * valid3
            if layer < 2:
                halo_ref[layer + 1] = h[41]
        g = jnp.sum(h[:NSHARE], axis=0)
        wsc_ref[NE, pl.ds(b, 1)] = _dot(g, wo_ref[...])[None]

    @pl.when(e > 0)
    def _window():
        tok2 = tokB2_ref[0, 0]
        tok3 = tokB3_ref[0, 0]
        feat, valid3 = _embed(tok2, tok3, embpeB_ref[...])
        rows = WL * BT
        z1 = jnp.zeros((1, BT, ND), jnp.float32)
        h = feat
        for layer in range(3):
            left = jnp.concatenate([halo_ref[layer][None], h[:-1]], axis=0)
            right = jnp.concatenate([h[1:], z1], axis=0)
            mi = left + right
            cat = jnp.concatenate([h.reshape(rows, ND), mi.reshape(rows, ND)],
                                  axis=1)
            pre = _dot(cat, wcat) + bc
            h = jnp.maximum(pre, 0.0).reshape(WL, BT, ND) * valid3
        g = jnp.sum(h, axis=0)
        wsc_ref[pl.ds(e - 1, 1), pl.ds(b, 1)] = _dot(g, wo_ref[...])[None, None]

    @pl.when((b == NB - 1) & (e == NE))
    def _route():
        for j in range(NB):
            preds = (wsc_ref[:NE, j, :, 0] + wsc_ref[NE:, j, :, 0]
                     + bo_ref[0, 0])
            pres = tok0_ref[:, j * BT:(j + 1) * BT] != 0
            masked = jnp.where(pres, preds, -1000.0)
            m = jnp.max(masked, axis=0, keepdims=True)
            ismax = masked == m
            ei = jax.lax.broadcasted_iota(jnp.int32, (NE, BT), 0)
            idx = jnp.min(jnp.where(ismax, ei, NE), axis=0, keepdims=True)
            sel = (ei == idx).astype(jnp.float32)
            out_ref[:, j * BT:(j + 1) * BT] = jnp.sum(
                preds * sel, axis=0, keepdims=True)


def kernel(x_data, emb, W_msg, b_msg, W_upd, b_upd, W_out, b_out):
    x_data = x_data.astype(jnp.int32)
    tok_core = x_data[:, :CORE]
    tok48 = jnp.pad(tok_core, ((0, 0), (0, CPAD - CORE)))
    tokA3 = tok48.T.reshape(CPAD, NB, BT).transpose(1, 0, 2)
    tokA2 = tokA3.reshape(NB, CPAD * BT, 1)

    tok_mhc = x_data[:, CORE:].reshape(BSZ, NE, MHCL).transpose(1, 0, 2)
    tok_win = jnp.concatenate(
        [jnp.broadcast_to(tok_core[None, :, W0:CORE], (NE, BSZ, CORE - W0)),
         tok_mhc], axis=2)
    tokB3 = (tok_win.transpose(0, 2, 1)
             .reshape(NE, WL, NB, BT).transpose(0, 2, 1, 3))
    tokB2 = tokB3.reshape(NE, NB, WL * BT, 1)
    tok0 = tok_mhc[:, :, 0]

    wcat = jnp.concatenate([W_upd, W_msg], axis=0)
    bc = (b_upd + b_msg).reshape(1, ND)
    bo = b_out.reshape(1, 1)
    embpe_a = jnp.concatenate([emb, jnp.asarray(_PE_CORE)], axis=0)
    embpe_b = jnp.concatenate([emb, jnp.asarray(_PE_WIN)], axis=0)

    ew = lambda b, e: (jnp.maximum(e - 1, 0), b, 0, 0)
    out = pl.pallas_call(
        _body,
        grid=(NB, NE + 1),
        in_specs=[
            pl.BlockSpec((1, CPAD * BT, 1), lambda b, e: (b, 0, 0)),
            pl.BlockSpec((1, CPAD, BT), lambda b, e: (b, 0, 0)),
            pl.BlockSpec((1, 1, WL * BT, 1), ew),
            pl.BlockSpec((1, 1, WL, BT), ew),
            pl.BlockSpec((NE, BSZ), lambda b, e: (0, 0)),
            pl.BlockSpec((1, 1), lambda b, e: (0, 0)),
            pl.BlockSpec((NV + CPAD, ND), lambda b, e: (0, 0)),
            pl.BlockSpec((ND2, ND), lambda b, e: (0, 0)),
            pl.BlockSpec((1, ND), lambda b, e: (0, 0)),
            pl.BlockSpec((ND, 1), lambda b, e: (0, 0)),
            pl.BlockSpec((NV + WL, ND), lambda b, e: (0, 0)),
        ],
        out_specs=pl.BlockSpec((1, BSZ), lambda b, e: (0, 0)),
        out_shape=jax.ShapeDtypeStruct((1, BSZ), jnp.float32),
        scratch_shapes=[
            pltpu.VMEM((NE + 1, NB, BT, 1), jnp.float32),
            pltpu.VMEM((3, BT, ND), jnp.float32),
        ],
    )(tokA2, tokA3, tokB2, tokB3, tok0, bo, embpe_a, wcat, bc, W_out,
      embpe_b)
    return out.reshape(BSZ)


# speedup vs baseline: 1.0061x; 1.0061x over previous
import numpy as np
import jax
import jax.numpy as jnp
from jax.experimental import pallas as pl
from jax.experimental.pallas import tpu as pltpu

ND = 256
ND2 = 512
NV = 30
NE = 16
BSZ = 256
CORE = 45
CPAD = 48
MHCL = 34
W0 = 42
WL = 37
LP = 79
NSHARE = 42
BT = 128
NB = BSZ // BT


def _pe_table(d, length):
    pos = np.arange(length)[:, None].astype(np.float32)
    i = np.arange(d)[None, :].astype(np.float32)
    ang = pos / np.power(10000.0, (2.0 * np.floor(i / 2.0)) / d)
    pe = np.zeros((length, d), np.float32)
    pe[:, 0::2] = np.sin(ang[:, 0::2])
    pe[:, 1::2] = np.cos(ang[:, 1::2])
    return pe


_PE79 = _pe_table(ND, LP)
_PE_CORE = np.zeros((CPAD, ND), np.float32)
_PE_CORE[:CORE] = _PE79[:CORE]
_PE_WIN = np.ascontiguousarray(_PE79[W0:LP])


def _dot(a, b):
    return jax.lax.dot_general(a, b, (((1,), (0,)), ((), ())),
                               preferred_element_type=jnp.float32)


def _embed(tok2, tok3, emb, pe):
    n, bt = tok3.shape
    rows = n * bt
    oh = (tok2 == jax.lax.broadcasted_iota(jnp.int32, (rows, NV), 1)
          ).astype(jnp.float32)
    feat2 = _dot(oh, emb)
    valid3 = (tok3 != 0).astype(jnp.float32)[:, :, None]
    feat = (feat2.reshape(n, bt, ND) + pe[:, None, :]) * valid3
    return feat, valid3


def _body(tokA2_ref, tokA3_ref, tokB2_ref, tokB3_ref, tok0_ref, bo_ref,
          emb_ref, wcat_ref, bc_ref, wo_ref, peA_ref, peB_ref,
          out_ref, wsc_ref, halo_ref):
    b = pl.program_id(0)
    e = pl.program_id(1)
    wcat = wcat_ref[...]
    bc = bc_ref[...]

    @pl.when(e == 0)
    def _core():
        tok2 = tokA2_ref[0]
        tok3 = tokA3_ref[0]
        feat, valid3 = _embed(tok2, tok3, emb_ref[...], peA_ref[...])
        rows = CPAD * BT
        halo_ref[0] = feat[41]
        z1 = jnp.zeros((1, BT, ND), jnp.float32)
        h = feat
        for layer in range(3):
            left = jnp.concatenate([z1, h[:-1]], axis=0)
            right = jnp.concatenate([h[1:], z1], axis=0)
            mi = left + right
            cat = jnp.concatenate([h.reshape(rows, ND), mi.reshape(rows, ND)],
                                  axis=1)
            pre = _dot(cat, wcat) + bc
            h = jnp.maximum(pre, 0.0).reshape(CPAD, BT, ND) * valid3
            if layer < 2:
                halo_ref[layer + 1] = h[41]
        g = jnp.sum(h[:NSHARE], axis=0)
        wsc_ref[NE, pl.ds(b, 1)] = _dot(g, wo_ref[...])[None]

    @pl.when(e > 0)
    def _window():
        tok2 = tokB2_ref[0, 0]
        tok3 = tokB3_ref[0, 0]
        feat, valid3 = _embed(tok2, tok3, emb_ref[...], peB_ref[...])
        rows = WL * BT
        z1 = jnp.zeros((1, BT, ND), jnp.float32)
        h = feat
        for layer in range(3):
            left = jnp.concatenate([halo_ref[layer][None], h[:-1]], axis=0)
            right = jnp.concatenate([h[1:], z1], axis=0)
            mi = left + right
            cat = jnp.concatenate([h.reshape(rows, ND), mi.reshape(rows, ND)],
                                  axis=1)
            pre = _dot(cat, wcat) + bc
            h = jnp.maximum(pre, 0.0).reshape(WL, BT, ND) * valid3
        g = jnp.sum(h, axis=0)
        wsc_ref[pl.ds(e - 1, 1), pl.ds(b, 1)] = _dot(g, wo_ref[...])[None, None]

    @pl.when((b == NB - 1) & (e == NE))
    def _route():
        for j in range(NB):
            preds = (wsc_ref[:NE, j, :, 0] + wsc_ref[NE:, j, :, 0]
                     + bo_ref[0, 0])
            pres = tok0_ref[:, j * BT:(j + 1) * BT] != 0
            masked = jnp.where(pres, preds, -1000.0)
            m = jnp.max(masked, axis=0, keepdims=True)
            ismax = masked == m
            ei = jax.lax.broadcasted_iota(jnp.int32, (NE, BT), 0)
            idx = jnp.min(jnp.where(ismax, ei, NE), axis=0, keepdims=True)
            sel = (ei == idx).astype(jnp.float32)
            out_ref[:, j * BT:(j + 1) * BT] = jnp.sum(
                preds * sel, axis=0, keepdims=True)


def kernel(x_data, emb, W_msg, b_msg, W_upd, b_upd, W_out, b_out):
    x_data = x_data.astype(jnp.int32)
    tok_core = x_data[:, :CORE]
    tok48 = jnp.pad(tok_core, ((0, 0), (0, CPAD - CORE)))
    tokA3 = tok48.T.reshape(CPAD, NB, BT).transpose(1, 0, 2)
    tokA2 = tokA3.reshape(NB, CPAD * BT, 1)

    tok_mhc = x_data[:, CORE:].reshape(BSZ, NE, MHCL).transpose(1, 0, 2)
    tok_win = jnp.concatenate(
        [jnp.broadcast_to(tok_core[None, :, W0:CORE], (NE, BSZ, CORE - W0)),
         tok_mhc], axis=2)
    tokB3 = (tok_win.transpose(0, 2, 1)
             .reshape(NE, WL, NB, BT).transpose(0, 2, 1, 3))
    tokB2 = tokB3.reshape(NE, NB, WL * BT, 1)
    tok0 = tok_mhc[:, :, 0]

    wcat = jnp.concatenate([W_upd, W_msg], axis=0)
    bc = (b_upd + b_msg).reshape(1, ND)
    bo = b_out.reshape(1, 1)
    pe_core = jnp.asarray(_PE_CORE)
    pe_win = jnp.asarray(_PE_WIN)

    ew = lambda b, e: (jnp.maximum(e - 1, 0), b, 0, 0)
    out = pl.pallas_call(
        _body,
        grid=(NB, NE + 1),
        in_specs=[
            pl.BlockSpec((1, CPAD * BT, 1), lambda b, e: (b, 0, 0)),
            pl.BlockSpec((1, CPAD, BT), lambda b, e: (b, 0, 0)),
            pl.BlockSpec((1, 1, WL * BT, 1), ew),
            pl.BlockSpec((1, 1, WL, BT), ew),
            pl.BlockSpec((NE, BSZ), lambda b, e: (0, 0)),
            pl.BlockSpec((1, 1), lambda b, e: (0, 0)),
            pl.BlockSpec((NV, ND), lambda b, e: (0, 0)),
            pl.BlockSpec((ND2, ND), lambda b, e: (0, 0)),
            pl.BlockSpec((1, ND), lambda b, e: (0, 0)),
            pl.BlockSpec((ND, 1), lambda b, e: (0, 0)),
            pl.BlockSpec((CPAD, ND), lambda b, e: (0, 0)),
            pl.BlockSpec((WL, ND), lambda b, e: (0, 0)),
        ],
        out_specs=pl.BlockSpec((1, BSZ), lambda b, e: (0, 0)),
        out_shape=jax.ShapeDtypeStruct((1, BSZ), jnp.float32),
        scratch_shapes=[
            pltpu.VMEM((NE + 1, NB, BT, 1), jnp.float32),
            pltpu.VMEM((3, BT, ND), jnp.float32),
        ],
    )(tokA2, tokA3, tokB2, tokB3, tok0, bo, emb, wcat, bc, W_out,
      pe_core, pe_win)
    return out.reshape(BSZ)
